# baseline (speedup 1.0000x reference)
import jax
import jax.numpy as jnp
from jax.experimental import pallas as pl
from jax.experimental.pallas import tpu as pltpu

_HID = 512
_HEADS = 8
_C = _HID // _HEADS
_L = 2
_NEG = 0.2
_N = 64
_F = 2


def _gatv2_frame_kernel(x_ref, bias_ref, aux_ref, win_ref, bin_ref, wl_ref,
                        bl_ref, wr_ref, br_ref, atte_ref, atto_ref,
                        bout_ref, lnw_ref, lnb_ref, out_ref):
    xf = x_ref[...].reshape(_F * _N, x_ref.shape[-1])
    h0 = jnp.dot(xf, win_ref[...], preferred_element_type=jnp.float32)
    h0 = h0 + bin_ref[...]
    mkcol = jnp.concatenate(
        [aux_ref[f, :, 0:1] for f in range(_F)], axis=0)

    hi = h0
    for li in range(_L):
        res = hi
        xl = jnp.dot(hi, wl_ref[li], preferred_element_type=jnp.float32)
        xl = xl + bl_ref[li:li + 1, :]
        xr = jnp.dot(hi, wr_ref[li], preferred_element_type=jnp.float32)
        xr = xr + br_ref[li:li + 1, :]
        frames_out = []
        for f in range(_F):
            r0 = f * _N
            bias = bias_ref[f]
            heads_out = []
            for hp in range(_HEADS // 2):
                lo = 2 * _C * hp
                xl2 = xl[r0:r0 + _N, lo:lo + 2 * _C]
                xr2 = xr[r0:r0 + _N, lo:lo + 2 * _C]
                ae = atte_ref[li:li + 1, lo:lo + 2 * _C].reshape(1, 1, 2 * _C)
                ao = atto_ref[li:li + 1, lo:lo + 2 * _C].reshape(1, 1, 2 * _C)
                xr8 = jnp.broadcast_to(xr2[:, None, :], (_N, 8, 2 * _C))
                xrb = jnp.broadcast_to(
                    xr8[:, None, :, :],
                    (_N, _N // 8, 8, 2 * _C)).reshape(_N, _N, 2 * _C)
                u = xrb + xl2[None, :, :]
                w = jnp.maximum(u, _NEG * u)
                e0 = jnp.sum(w * ae, axis=-1) + bias
                e1 = jnp.sum(w * ao, axis=-1) + bias
                for e, c0 in ((e0, lo), (e1, lo + _C)):
                    m = jnp.max(e, axis=-1, keepdims=True)
                    p = jnp.exp(e - m)
                    alpha = p * (1.0 / jnp.sum(p, axis=-1, keepdims=True))
                    heads_out.append(jnp.dot(
                        alpha, xl[r0:r0 + _N, c0:c0 + _C],
                        preferred_element_type=jnp.float32))
            frames_out.append(jnp.concatenate(heads_out, axis=-1))
        out = jnp.concatenate(frames_out, axis=0)
        out = out + bout_ref[li:li + 1, :]
        out = jnp.where(out > 0.0, out, jnp.exp(out) - 1.0)
        mu = jnp.mean(out, axis=-1, keepdims=True)
        cen = out - mu
        var = jnp.mean(cen * cen, axis=-1, keepdims=True)
        out = cen * jax.lax.rsqrt(var + 1e-5)
        out = out * lnw_ref[li:li + 1, :] + lnb_ref[li:li + 1, :]
        out = out + res
        out = out * mkcol
        hi = out
    conds = jnp.concatenate(
        [jnp.broadcast_to(aux_ref[f, 0:1, 1:2], (_N, 1)) for f in range(_F)],
        axis=0)
    fin = jnp.where(conds > 0.5, h0, hi)
    out_ref[...] = fin.reshape(_F, _N, _HID)


def kernel(x, person_mask, W_in, b_in, Wl, bl, Wr, br, att, b_out, ln_w, ln_b):
    Bb, Tt, Nn, Dd = x.shape
    bt = Bb * Tt
    x3 = x.reshape(bt, Nn, Dd)
    m = person_mask.reshape(bt, Nn)
    mf = m.astype(jnp.float32)

    eye = jnp.eye(Nn, dtype=bool)
    allowed = (m[:, :, None] & m[:, None, :]) | eye[None]
    bias = jnp.where(allowed, 0.0, -1e9).astype(jnp.float32)

    cond = (jnp.sum(mf, axis=1) <= 1.0).astype(jnp.float32)
    aux = jnp.zeros((bt, Nn, 8), jnp.float32)
    aux = aux.at[:, :, 0].set(mf)
    aux = aux.at[:, :, 1].set(cond[:, None])

    att2 = att.reshape(_L, _HID)
    lane = jnp.arange(_HID)
    even = ((lane // _C) % 2 == 0).astype(jnp.float32)
    attE = att2 * even[None, :]
    attO = att2 * (1.0 - even)[None, :]
    b_in2 = b_in.reshape(1, _HID)

    grid = (bt // _F,)
    out = pl.pallas_call(
        _gatv2_frame_kernel,
        grid=grid,
        in_specs=[
            pl.BlockSpec((_F, Nn, Dd), lambda f: (f, 0, 0)),
            pl.BlockSpec((_F, Nn, Nn), lambda f: (f, 0, 0)),
            pl.BlockSpec((_F, Nn, 8), lambda f: (f, 0, 0)),
            pl.BlockSpec((Dd, _HID), lambda f: (0, 0)),
            pl.BlockSpec((1, _HID), lambda f: (0, 0)),
            pl.BlockSpec((_L, _HID, _HID), lambda f: (0, 0, 0)),
            pl.BlockSpec((_L, _HID), lambda f: (0, 0)),
            pl.BlockSpec((_L, _HID, _HID), lambda f: (0, 0, 0)),
            pl.BlockSpec((_L, _HID), lambda f: (0, 0)),
            pl.BlockSpec((_L, _HID), lambda f: (0, 0)),
            pl.BlockSpec((_L, _HID), lambda f: (0, 0)),
            pl.BlockSpec((_L, _HID), lambda f: (0, 0)),
            pl.BlockSpec((_L, _HID), lambda f: (0, 0)),
            pl.BlockSpec((_L, _HID), lambda f: (0, 0)),
        ],
        out_specs=pl.BlockSpec((_F, Nn, _HID), lambda f: (f, 0, 0)),
        out_shape=jax.ShapeDtypeStruct((bt, Nn, _HID), jnp.float32),
        compiler_params=pltpu.CompilerParams(
            dimension_semantics=("parallel",),
        ),
    )(x3, bias, aux, W_in, b_in2, Wl, bl, Wr, br, attE, attO, b_out,
      ln_w, ln_b)
    return out.reshape(Bb, Tt, Nn, _HID)


# speedup vs baseline: 3.7188x; 3.7188x over previous
import jax
import jax.numpy as jnp
from jax.experimental import pallas as pl
from jax.experimental.pallas import tpu as pltpu

_HID = 512
_HEADS = 8
_C = _HID // _HEADS
_L = 2
_NEG = 0.2
_N = 64
_F = 2


def _gatv2_frame_kernel(x_ref, bias_ref, aux_ref, win_ref, bin_ref, wl_ref,
                        bl_ref, wr_ref, br_ref, atte_ref, atto_ref,
                        bout_ref, lnw_ref, lnb_ref, out_ref):
    xf = x_ref[...].reshape(_F * _N, x_ref.shape[-1])
    h0 = jnp.dot(xf, win_ref[...], preferred_element_type=jnp.float32)
    h0 = h0 + bin_ref[...]
    bias2s = [jnp.concatenate([bias_ref[f], bias_ref[f]], axis=-1)
              for f in range(_F)]
    mkcol = jnp.concatenate(
        [aux_ref[f, :, 0:1] for f in range(_F)], axis=0)

    hi = h0
    for li in range(_L):
        res = hi
        xl = jnp.dot(hi, wl_ref[li], preferred_element_type=jnp.float32)
        xl = xl + bl_ref[li:li + 1, :]
        xr = jnp.dot(hi, wr_ref[li], preferred_element_type=jnp.float32)
        xr = xr + br_ref[li:li + 1, :]
        frames_out = []
        for f in range(_F):
            r0 = f * _N
            bias2 = bias2s[f]
            heads_out = []
            for hp in range(_HEADS // 2):
                lo = 2 * _C * hp
                xl2 = xl[r0:r0 + _N, lo:lo + 2 * _C]
                xr2 = xr[r0:r0 + _N, lo:lo + 2 * _C]
                ae = atte_ref[li:li + 1, lo:lo + 2 * _C].reshape(1, 1, 2 * _C)
                ao = atto_ref[li:li + 1, lo:lo + 2 * _C].reshape(1, 1, 2 * _C)
                u = xr2[:, None, :] + xl2[None, :, :]
                w = jnp.maximum(u, _NEG * u)
                e0 = jnp.sum(w * ae, axis=-1)
                e1 = jnp.sum(w * ao, axis=-1)
                e2 = jnp.concatenate([e0, e1], axis=-1) + bias2
                m0 = jnp.max(e2[:, :_N], axis=-1, keepdims=True)
                m1 = jnp.max(e2[:, _N:], axis=-1, keepdims=True)
                m2 = jnp.concatenate([jnp.broadcast_to(m0, (_N, _N)),
                                      jnp.broadcast_to(m1, (_N, _N))], axis=-1)
                p2 = jnp.exp(e2 - m2)
                z0 = 1.0 / jnp.sum(p2[:, :_N], axis=-1, keepdims=True)
                z1 = 1.0 / jnp.sum(p2[:, _N:], axis=-1, keepdims=True)
                z2 = jnp.concatenate([jnp.broadcast_to(z0, (_N, _N)),
                                      jnp.broadcast_to(z1, (_N, _N))], axis=-1)
                alpha2 = p2 * z2
                heads_out.append(jnp.dot(
                    alpha2[:, :_N], xl[r0:r0 + _N, lo:lo + _C],
                    preferred_element_type=jnp.float32))
                heads_out.append(jnp.dot(
                    alpha2[:, _N:], xl[r0:r0 + _N, lo + _C:lo + 2 * _C],
                    preferred_element_type=jnp.float32))
            frames_out.append(jnp.concatenate(heads_out, axis=-1))
        out = jnp.concatenate(frames_out, axis=0)
        out = out + bout_ref[li:li + 1, :]
        out = jnp.where(out > 0.0, out, jnp.exp(out) - 1.0)
        mu = jnp.mean(out, axis=-1, keepdims=True)
        cen = out - mu
        var = jnp.mean(cen * cen, axis=-1, keepdims=True)
        out = cen * jax.lax.rsqrt(var + 1e-5)
        out = out * lnw_ref[li:li + 1, :] + lnb_ref[li:li + 1, :]
        out = out + res
        out = out * mkcol
        hi = out
    conds = jnp.concatenate(
        [jnp.broadcast_to(aux_ref[f, 0:1, 1:2], (_N, 1)) for f in range(_F)],
        axis=0)
    fin = jnp.where(conds > 0.5, h0, hi)
    out_ref[...] = fin.reshape(_F, _N, _HID)


def kernel(x, person_mask, W_in, b_in, Wl, bl, Wr, br, att, b_out, ln_w, ln_b):
    Bb, Tt, Nn, Dd = x.shape
    bt = Bb * Tt
    x3 = x.reshape(bt, Nn, Dd)
    m = person_mask.reshape(bt, Nn)
    mf = m.astype(jnp.float32)

    eye = jnp.eye(Nn, dtype=bool)
    allowed = (m[:, :, None] & m[:, None, :]) | eye[None]
    bias = jnp.where(allowed, 0.0, -1e9).astype(jnp.float32)

    cond = (jnp.sum(mf, axis=1) <= 1.0).astype(jnp.float32)
    aux = jnp.zeros((bt, Nn, 8), jnp.float32)
    aux = aux.at[:, :, 0].set(mf)
    aux = aux.at[:, :, 1].set(cond[:, None])

    att2 = att.reshape(_L, _HID)
    lane = jnp.arange(_HID)
    even = ((lane // _C) % 2 == 0).astype(jnp.float32)
    attE = att2 * even[None, :]
    attO = att2 * (1.0 - even)[None, :]
    b_in2 = b_in.reshape(1, _HID)

    grid = (bt // _F,)
    out = pl.pallas_call(
        _gatv2_frame_kernel,
        grid=grid,
        in_specs=[
            pl.BlockSpec((_F, Nn, Dd), lambda f: (f, 0, 0)),
            pl.BlockSpec((_F, Nn, Nn), lambda f: (f, 0, 0)),
            pl.BlockSpec((_F, Nn, 8), lambda f: (f, 0, 0)),
            pl.BlockSpec((Dd, _HID), lambda f: (0, 0)),
            pl.BlockSpec((1, _HID), lambda f: (0, 0)),
            pl.BlockSpec((_L, _HID, _HID), lambda f: (0, 0, 0)),
            pl.BlockSpec((_L, _HID), lambda f: (0, 0)),
            pl.BlockSpec((_L, _HID, _HID), lambda f: (0, 0, 0)),
            pl.BlockSpec((_L, _HID), lambda f: (0, 0)),
            pl.BlockSpec((_L, _HID), lambda f: (0, 0)),
            pl.BlockSpec((_L, _HID), lambda f: (0, 0)),
            pl.BlockSpec((_L, _HID), lambda f: (0, 0)),
            pl.BlockSpec((_L, _HID), lambda f: (0, 0)),
            pl.BlockSpec((_L, _HID), lambda f: (0, 0)),
        ],
        out_specs=pl.BlockSpec((_F, Nn, _HID), lambda f: (f, 0, 0)),
        out_shape=jax.ShapeDtypeStruct((bt, Nn, _HID), jnp.float32),
        compiler_params=pltpu.CompilerParams(
            dimension_semantics=("parallel",),
        ),
    )(x3, bias, aux, W_in, b_in2, Wl, bl, Wr, br, attE, attO, b_out,
      ln_w, ln_b)
    return out.reshape(Bb, Tt, Nn, _HID)


# speedup vs baseline: 3.7716x; 1.0142x over previous
import jax
import jax.numpy as jnp
from jax.experimental import pallas as pl
from jax.experimental.pallas import tpu as pltpu

_HID = 512
_HEADS = 8
_C = _HID // _HEADS
_L = 2
_NEG = 0.2
_N = 64
_F = 2


def _gatv2_frame_kernel(x_ref, bias_ref, aux_ref, win_ref, bin_ref, wl_ref,
                        bl_ref, wr_ref, br_ref, atte_ref, atto_ref,
                        bout_ref, lnw_ref, lnb_ref, out_ref):
    xf = x_ref[...].reshape(_F * _N, x_ref.shape[-1])
    h0 = jnp.dot(xf, win_ref[...], preferred_element_type=jnp.float32)
    h0 = h0 + bin_ref[...]
    bias2s = [jnp.concatenate([bias_ref[f], bias_ref[f]], axis=-1)
              for f in range(_F)]
    mkcol = jnp.concatenate(
        [aux_ref[f, :, 0:1] for f in range(_F)], axis=0)

    hi = h0
    for li in range(_L):
        res = hi
        xl = jnp.dot(hi, wl_ref[li], preferred_element_type=jnp.float32)
        xl = xl + bl_ref[li:li + 1, :]
        xr = jnp.dot(hi, wr_ref[li], preferred_element_type=jnp.float32)
        xr = xr + br_ref[li:li + 1, :]
        frames_heads = [[] for _ in range(_F)]
        for hp in range(_HEADS // 2):
            lo = 2 * _C * hp
            for f in range(_F):
                r0 = f * _N
                bias2 = bias2s[f]
                heads_out = frames_heads[f]
                xl2 = xl[r0:r0 + _N, lo:lo + 2 * _C]
                xr2 = xr[r0:r0 + _N, lo:lo + 2 * _C]
                ae = atte_ref[li:li + 1, lo:lo + 2 * _C].reshape(1, 1, 2 * _C)
                ao = atto_ref[li:li + 1, lo:lo + 2 * _C].reshape(1, 1, 2 * _C)
                u = xr2[:, None, :] + xl2[None, :, :]
                w = jnp.maximum(u, _NEG * u)
                e0 = jnp.sum(w * ae, axis=-1)
                e1 = jnp.sum(w * ao, axis=-1)
                e2 = jnp.concatenate([e0, e1], axis=-1) + bias2
                m0 = jnp.max(e2[:, :_N], axis=-1, keepdims=True)
                m1 = jnp.max(e2[:, _N:], axis=-1, keepdims=True)
                m2 = jnp.concatenate([jnp.broadcast_to(m0, (_N, _N)),
                                      jnp.broadcast_to(m1, (_N, _N))], axis=-1)
                p2 = jnp.exp(e2 - m2)
                z0 = 1.0 / jnp.sum(p2[:, :_N], axis=-1, keepdims=True)
                z1 = 1.0 / jnp.sum(p2[:, _N:], axis=-1, keepdims=True)
                z2 = jnp.concatenate([jnp.broadcast_to(z0, (_N, _N)),
                                      jnp.broadcast_to(z1, (_N, _N))], axis=-1)
                alpha2 = p2 * z2
                heads_out.append(jnp.dot(
                    alpha2[:, :_N], xl[r0:r0 + _N, lo:lo + _C],
                    preferred_element_type=jnp.float32))
                heads_out.append(jnp.dot(
                    alpha2[:, _N:], xl[r0:r0 + _N, lo + _C:lo + 2 * _C],
                    preferred_element_type=jnp.float32))
        out = jnp.concatenate(
            [jnp.concatenate(h, axis=-1) for h in frames_heads], axis=0)
        out = out + bout_ref[li:li + 1, :]
        out = jnp.where(out > 0.0, out, jnp.exp(out) - 1.0)
        mu = jnp.mean(out, axis=-1, keepdims=True)
        cen = out - mu
        var = jnp.mean(cen * cen, axis=-1, keepdims=True)
        out = cen * jax.lax.rsqrt(var + 1e-5)
        out = out * lnw_ref[li:li + 1, :] + lnb_ref[li:li + 1, :]
        out = out + res
        out = out * mkcol
        hi = out
    conds = jnp.concatenate(
        [jnp.broadcast_to(aux_ref[f, 0:1, 1:2], (_N, 1)) for f in range(_F)],
        axis=0)
    fin = jnp.where(conds > 0.5, h0, hi)
    out_ref[...] = fin.reshape(_F, _N, _HID)


def kernel(x, person_mask, W_in, b_in, Wl, bl, Wr, br, att, b_out, ln_w, ln_b):
    Bb, Tt, Nn, Dd = x.shape
    bt = Bb * Tt
    x3 = x.reshape(bt, Nn, Dd)
    m = person_mask.reshape(bt, Nn)
    mf = m.astype(jnp.float32)

    eye = jnp.eye(Nn, dtype=bool)
    allowed = (m[:, :, None] & m[:, None, :]) | eye[None]
    bias = jnp.where(allowed, 0.0, -1e9).astype(jnp.float32)

    cond = (jnp.sum(mf, axis=1) <= 1.0).astype(jnp.float32)
    aux = jnp.zeros((bt, Nn, 8), jnp.float32)
    aux = aux.at[:, :, 0].set(mf)
    aux = aux.at[:, :, 1].set(cond[:, None])

    att2 = att.reshape(_L, _HID)
    lane = jnp.arange(_HID)
    even = ((lane // _C) % 2 == 0).astype(jnp.float32)
    attE = att2 * even[None, :]
    attO = att2 * (1.0 - even)[None, :]
    b_in2 = b_in.reshape(1, _HID)

    grid = (bt // _F,)
    out = pl.pallas_call(
        _gatv2_frame_kernel,
        grid=grid,
        in_specs=[
            pl.BlockSpec((_F, Nn, Dd), lambda f: (f, 0, 0)),
            pl.BlockSpec((_F, Nn, Nn), lambda f: (f, 0, 0)),
            pl.BlockSpec((_F, Nn, 8), lambda f: (f, 0, 0)),
            pl.BlockSpec((Dd, _HID), lambda f: (0, 0)),
            pl.BlockSpec((1, _HID), lambda f: (0, 0)),
            pl.BlockSpec((_L, _HID, _HID), lambda f: (0, 0, 0)),
            pl.BlockSpec((_L, _HID), lambda f: (0, 0)),
            pl.BlockSpec((_L, _HID, _HID), lambda f: (0, 0, 0)),
            pl.BlockSpec((_L, _HID), lambda f: (0, 0)),
            pl.BlockSpec((_L, _HID), lambda f: (0, 0)),
            pl.BlockSpec((_L, _HID), lambda f: (0, 0)),
            pl.BlockSpec((_L, _HID), lambda f: (0, 0)),
            pl.BlockSpec((_L, _HID), lambda f: (0, 0)),
            pl.BlockSpec((_L, _HID), lambda f: (0, 0)),
        ],
        out_specs=pl.BlockSpec((_F, Nn, _HID), lambda f: (f, 0, 0)),
        out_shape=jax.ShapeDtypeStruct((bt, Nn, _HID), jnp.float32),
        compiler_params=pltpu.CompilerParams(
            dimension_semantics=("parallel",),
        ),
    )(x3, bias, aux, W_in, b_in2, Wl, bl, Wr, br, attE, attO, b_out,
      ln_w, ln_b)
    return out.reshape(Bb, Tt, Nn, _HID)
